# baseline (speedup 1.0000x reference)
import functools
import math

import jax
import jax.numpy as jnp
from jax import lax
from jax.experimental import pallas as pl
from jax.experimental.pallas import tpu as pltpu
from jax.experimental.pallas import tpu_sc as plsc


def kernel(x, pos_table):
    batch, seq_len, d_model = x.shape
    scale = jnp.float32(math.sqrt(d_model))
    info = plsc.get_sparse_core_info()
    nc, ns, lanes = info.num_cores, info.num_subcores, info.num_lanes
    nw = nc * ns
    s_per_w = seq_len // nw
    ch = 16
    n_chunks = s_per_w // ch
    chw = ch * d_model

    mesh = plsc.VectorSubcoreMesh(core_axis_name="c", subcore_axis_name="s")

    @functools.partial(
        pl.kernel,
        mesh=mesh,
        out_type=jax.ShapeDtypeStruct((batch, seq_len * d_model), jnp.float32),
        scratch_types=[
            pltpu.VMEM((chw,), jnp.float32),
            pltpu.VMEM((chw,), jnp.float32),
        ],
    )
    def sc_k(x_hbm, pos_hbm, out_hbm, pos_v, x_v):
        wid = lax.axis_index("s") * nc + lax.axis_index("c")
        base = wid * s_per_w * d_model

        def chunk_body(c, carry):
            off = base + c * chw
            pltpu.sync_copy(pos_hbm.at[pl.ds(off, chw)], pos_v)
            for b in range(batch):
                pltpu.sync_copy(x_hbm.at[b, pl.ds(off, chw)], x_v)

                def vec_body(i, carry2):
                    sl = pl.ds(i * lanes, lanes)
                    x_v[sl] = x_v[sl] * scale + pos_v[sl]
                    return carry2

                lax.fori_loop(0, chw // lanes, vec_body, 0, unroll=4)
                pltpu.sync_copy(x_v, out_hbm.at[b, pl.ds(off, chw)])
            return carry

        lax.fori_loop(0, n_chunks, chunk_body, 0)

    x2 = x.reshape(batch, seq_len * d_model)
    pos2 = pos_table[:seq_len].reshape(seq_len * d_model)
    out = sc_k(x2, pos2)
    return out.reshape(batch, seq_len, d_model)


# speedup vs baseline: 1.2838x; 1.2838x over previous
import functools
import math

import jax
import jax.numpy as jnp
from jax import lax
from jax.experimental import pallas as pl
from jax.experimental.pallas import tpu as pltpu
from jax.experimental.pallas import tpu_sc as plsc


def kernel(x, pos_table):
    batch, seq_len, d_model = x.shape
    scale = jnp.float32(math.sqrt(d_model))
    info = plsc.get_sparse_core_info()
    nc, ns, lanes = info.num_cores, info.num_subcores, info.num_lanes
    nw = nc * ns
    s_per_w = seq_len // nw
    ch = 8
    n_chunks = s_per_w // ch
    assert n_chunks % 2 == 0 and s_per_w % ch == 0
    chw = ch * d_model
    n_vec = chw // lanes

    mesh = plsc.VectorSubcoreMesh(core_axis_name="c", subcore_axis_name="s")

    scratch = (
        [pltpu.VMEM((chw,), jnp.float32) for _ in range(2 * batch)]
        + [pltpu.VMEM((chw,), jnp.float32) for _ in range(2)]
        + [pltpu.SemaphoreType.DMA for _ in range(2 * batch)]
        + [pltpu.SemaphoreType.DMA for _ in range(2 * batch)]
        + [pltpu.SemaphoreType.DMA for _ in range(2)]
    )

    @functools.partial(
        pl.kernel,
        mesh=mesh,
        out_type=jax.ShapeDtypeStruct((batch, seq_len * d_model), jnp.float32),
        scratch_types=scratch,
    )
    def sc_k(x_hbm, pos_hbm, out_hbm, *refs):
        xi = [[refs[p * batch + b] for b in range(batch)] for p in range(2)]
        pp = [refs[2 * batch], refs[2 * batch + 1]]
        o = 2 * batch + 2
        sx = [[refs[o + p * batch + b] for b in range(batch)] for p in range(2)]
        o += 2 * batch
        so = [[refs[o + p * batch + b] for b in range(batch)] for p in range(2)]
        o += 2 * batch
        sp = [refs[o], refs[o + 1]]

        wid = lax.axis_index("s") * nc + lax.axis_index("c")
        base = wid * s_per_w * d_model

        pltpu.async_copy(pos_hbm.at[pl.ds(base, chw)], pp[0], sp[0])
        for b in range(batch):
            pltpu.async_copy(x_hbm.at[b, pl.ds(base, chw)], xi[0][b], sx[0][b])

        def pair_body(p, carry):
            for par in (0, 1):
                c = 2 * p + par
                off = base + c * chw
                nxt = off + chw

                @pl.when(c + 1 < n_chunks)
                def _():
                    pltpu.async_copy(
                        pos_hbm.at[pl.ds(nxt, chw)], pp[1 - par], sp[1 - par])

                pltpu.make_async_copy(
                    pos_hbm.at[pl.ds(off, chw)], pp[par], sp[par]).wait()

                for b in range(batch):
                    @pl.when(c >= 1)
                    def _():
                        pltpu.make_async_copy(
                            xi[1 - par][b], out_hbm.at[b, pl.ds(off, chw)],
                            so[1 - par][b]).wait()

                    @pl.when(c + 1 < n_chunks)
                    def _():
                        pltpu.async_copy(
                            x_hbm.at[b, pl.ds(nxt, chw)], xi[1 - par][b],
                            sx[1 - par][b])

                    pltpu.make_async_copy(
                        x_hbm.at[b, pl.ds(off, chw)], xi[par][b],
                        sx[par][b]).wait()

                    xr, pr = xi[par][b], pp[par]

                    def vec_body(i, c2):
                        sl = pl.ds(i * lanes, lanes)
                        xr[sl] = xr[sl] * scale + pr[sl]
                        return c2

                    lax.fori_loop(0, n_vec, vec_body, 0, unroll=8)
                    pltpu.async_copy(
                        xr, out_hbm.at[b, pl.ds(off, chw)], so[par][b])
            return carry

        lax.fori_loop(0, n_chunks // 2, pair_body, 0)

        last = base + (n_chunks - 1) * chw
        for b in range(batch):
            pltpu.make_async_copy(
                xi[1][b], out_hbm.at[b, pl.ds(last, chw)], so[1][b]).wait()

    x2 = x.reshape(batch, seq_len * d_model)
    pos2 = pos_table[:seq_len].reshape(seq_len * d_model)
    out = sc_k(x2, pos2)
    return out.reshape(batch, seq_len, d_model)


# speedup vs baseline: 2.0638x; 1.6075x over previous
import functools
import math

import jax
import jax.numpy as jnp
from jax import lax
from jax.experimental import pallas as pl
from jax.experimental.pallas import tpu as pltpu
from jax.experimental.pallas import tpu_sc as plsc


def kernel(x, pos_table):
    batch, seq_len, d_model = x.shape
    scale = jnp.float32(math.sqrt(d_model))
    info = plsc.get_sparse_core_info()
    nc, ns, lanes = info.num_cores, info.num_subcores, info.num_lanes
    nw = nc * ns
    s_per_w = seq_len // nw
    ch = 8
    n_chunks = s_per_w // ch
    assert n_chunks % 2 == 0 and s_per_w % ch == 0
    chw = ch * d_model
    n_vec = chw // lanes

    mesh = plsc.VectorSubcoreMesh(core_axis_name="c", subcore_axis_name="s")

    scratch = (
        [pltpu.VMEM((chw,), jnp.float32) for _ in range(2 * batch)]
        + [pltpu.VMEM((chw,), jnp.float32) for _ in range(2)]
        + [pltpu.SemaphoreType.DMA for _ in range(2 * batch)]
        + [pltpu.SemaphoreType.DMA for _ in range(2 * batch)]
        + [pltpu.SemaphoreType.DMA for _ in range(2)]
    )

    @functools.partial(
        pl.kernel,
        mesh=mesh,
        out_type=jax.ShapeDtypeStruct((batch, seq_len * d_model), jnp.float32),
        scratch_types=scratch,
    )
    def sc_k(x_hbm, pos_hbm, out_hbm, *refs):
        xi = [[refs[p * batch + b] for b in range(batch)] for p in range(2)]
        pp = [refs[2 * batch], refs[2 * batch + 1]]
        o = 2 * batch + 2
        sx = [[refs[o + p * batch + b] for b in range(batch)] for p in range(2)]
        o += 2 * batch
        so = [[refs[o + p * batch + b] for b in range(batch)] for p in range(2)]
        o += 2 * batch
        sp = [refs[o], refs[o + 1]]

        wid = lax.axis_index("s") * nc + lax.axis_index("c")
        base = wid * s_per_w * d_model

        pltpu.async_copy(pos_hbm.at[pl.ds(base, chw)], pp[0], sp[0])
        for b in range(batch):
            pltpu.async_copy(x_hbm.at[b, pl.ds(base, chw)], xi[0][b], sx[0][b])

        def pair_body(p, carry):
            for par in (0, 1):
                c = 2 * p + par
                off = base + c * chw
                nxt = off + chw

                @pl.when(c + 1 < n_chunks)
                def _():
                    pltpu.async_copy(
                        pos_hbm.at[pl.ds(nxt, chw)], pp[1 - par], sp[1 - par])

                pltpu.make_async_copy(
                    pos_hbm.at[pl.ds(off, chw)], pp[par], sp[par]).wait()

                for b in range(batch):
                    @pl.when(c >= 1)
                    def _():
                        pltpu.make_async_copy(
                            xi[1 - par][b], out_hbm.at[b, pl.ds(off, chw)],
                            so[1 - par][b]).wait()

                    @pl.when(c + 1 < n_chunks)
                    def _():
                        pltpu.async_copy(
                            x_hbm.at[b, pl.ds(nxt, chw)], xi[1 - par][b],
                            sx[1 - par][b])

                    pltpu.make_async_copy(
                        x_hbm.at[b, pl.ds(off, chw)], xi[par][b],
                        sx[par][b]).wait()

                    xr, pr = xi[par][b], pp[par]

                    def vec_body(g, c2):
                        i0 = g * (8 * lanes)
                        sls = [pl.ds(i0 + k * lanes, lanes) for k in range(8)]
                        xs = [xr[sl] for sl in sls]
                        ps = [pr[sl] for sl in sls]
                        rs = [xv * scale + pv for xv, pv in zip(xs, ps)]
                        for sl, rv in zip(sls, rs):
                            xr[sl] = rv
                        return c2

                    lax.fori_loop(0, n_vec // 8, vec_body, 0)
                    pltpu.async_copy(
                        xr, out_hbm.at[b, pl.ds(off, chw)], so[par][b])
            return carry

        lax.fori_loop(0, n_chunks // 2, pair_body, 0)

        last = base + (n_chunks - 1) * chw
        for b in range(batch):
            pltpu.make_async_copy(
                xi[1][b], out_hbm.at[b, pl.ds(last, chw)], so[1][b]).wait()

    x2 = x.reshape(batch, seq_len * d_model)
    pos2 = pos_table[:seq_len].reshape(seq_len * d_model)
    out = sc_k(x2, pos2)
    return out.reshape(batch, seq_len, d_model)


# speedup vs baseline: 5.5981x; 2.7125x over previous
import functools
import math

import jax
import jax.numpy as jnp
from jax import lax
from jax.experimental import pallas as pl
from jax.experimental.pallas import tpu as pltpu
from jax.experimental.pallas import tpu_sc as plsc


def kernel(x, pos_table):
    batch, seq_len, d_model = x.shape
    scale = jnp.float32(math.sqrt(d_model))
    info = plsc.get_sparse_core_info()
    nc, ns, lanes = info.num_cores, info.num_subcores, info.num_lanes
    nw = nc * ns
    s_per_w = seq_len // nw
    ch = 8
    n_chunks = s_per_w // ch
    assert n_chunks % 2 == 0 and s_per_w % ch == 0
    n_grp = d_model // (8 * lanes)

    mesh = plsc.VectorSubcoreMesh(core_axis_name="c", subcore_axis_name="s")

    scratch = (
        [pltpu.VMEM((ch, d_model), jnp.float32) for _ in range(2 * batch)]
        + [pltpu.VMEM((ch, d_model), jnp.float32) for _ in range(2)]
        + [pltpu.SemaphoreType.DMA for _ in range(2 * batch)]
        + [pltpu.SemaphoreType.DMA for _ in range(2 * batch)]
        + [pltpu.SemaphoreType.DMA for _ in range(2)]
    )

    @functools.partial(
        pl.kernel,
        mesh=mesh,
        out_type=jax.ShapeDtypeStruct((batch, seq_len, d_model), jnp.float32),
        scratch_types=scratch,
    )
    def sc_k(x_hbm, pos_hbm, out_hbm, *refs):
        xi = [[refs[p * batch + b] for b in range(batch)] for p in range(2)]
        pp = [refs[2 * batch], refs[2 * batch + 1]]
        o = 2 * batch + 2
        sx = [[refs[o + p * batch + b] for b in range(batch)] for p in range(2)]
        o += 2 * batch
        so = [[refs[o + p * batch + b] for b in range(batch)] for p in range(2)]
        o += 2 * batch
        sp = [refs[o], refs[o + 1]]

        wid = lax.axis_index("s") * nc + lax.axis_index("c")
        base = wid * s_per_w

        pltpu.async_copy(pos_hbm.at[pl.ds(base, ch), :], pp[0], sp[0])
        for b in range(batch):
            pltpu.async_copy(x_hbm.at[b, pl.ds(base, ch), :], xi[0][b], sx[0][b])

        def pair_body(p, carry):
            for par in (0, 1):
                c = 2 * p + par
                r0 = base + c * ch
                r1 = r0 + ch

                @pl.when(c + 1 < n_chunks)
                def _():
                    pltpu.async_copy(
                        pos_hbm.at[pl.ds(r1, ch), :], pp[1 - par], sp[1 - par])

                pltpu.make_async_copy(
                    pos_hbm.at[pl.ds(r0, ch), :], pp[par], sp[par]).wait()

                for b in range(batch):
                    @pl.when(c >= 1)
                    def _():
                        pltpu.make_async_copy(
                            xi[1 - par][b], out_hbm.at[b, pl.ds(r0, ch), :],
                            so[1 - par][b]).wait()

                    @pl.when(c + 1 < n_chunks)
                    def _():
                        pltpu.async_copy(
                            x_hbm.at[b, pl.ds(r1, ch), :], xi[1 - par][b],
                            sx[1 - par][b])

                    pltpu.make_async_copy(
                        x_hbm.at[b, pl.ds(r0, ch), :], xi[par][b],
                        sx[par][b]).wait()

                    xr, pr = xi[par][b], pp[par]

                    def row_body(r, c2):
                        def grp_body(g, c3):
                            i0 = g * (8 * lanes)
                            sls = [pl.ds(i0 + k * lanes, lanes)
                                   for k in range(8)]
                            xs = [xr[r, sl] for sl in sls]
                            ps = [pr[r, sl] for sl in sls]
                            rs = [xv * scale + pv
                                  for xv, pv in zip(xs, ps)]
                            for sl, rv in zip(sls, rs):
                                xr[r, sl] = rv
                            return c3

                        return lax.fori_loop(0, n_grp, grp_body, c2)

                    lax.fori_loop(0, ch, row_body, 0)
                    pltpu.async_copy(
                        xr, out_hbm.at[b, pl.ds(r0, ch), :], so[par][b])
            return carry

        lax.fori_loop(0, n_chunks // 2, pair_body, 0)

        rl = base + (n_chunks - 1) * ch
        for b in range(batch):
            pltpu.make_async_copy(
                xi[1][b], out_hbm.at[b, pl.ds(rl, ch), :], so[1][b]).wait()

    return sc_k(x, pos_table[:seq_len])
